# Initial kernel scaffold; baseline (speedup 1.0000x reference)
import jax
import jax.numpy as jnp
from jax.experimental import pallas as pl


def kernel(x, table):
    raise NotImplementedError("write your pallas kernel here")


# speedup vs baseline: 1.4724x; 1.4724x over previous
import functools

import jax
import jax.numpy as jnp
from jax import lax
from jax.experimental import pallas as pl
from jax.experimental.pallas import tpu as pltpu
from jax.experimental.pallas import tpu_sc as plsc

VOCAB = 1000000
EMBED_DIM = 32

NC = 2
NS = 16
NW = NC * NS

B_TOTAL = 4096 * 200
PER_W = B_TOTAL // NW
IDX_CHUNK = 128
K = 10
CHUNK = IDX_CHUNK * K
NCHUNK = PER_W // CHUNK


def _body(x_ref, table_ref, out_ref, idx_v, rows_v, sem):
    wid = lax.axis_index("s") * NC + lax.axis_index("c")

    @pl.loop(0, NCHUNK)
    def _chunk(c):
        pltpu.sync_copy(x_ref.at[wid, c], idx_v)
        copies = [
            pltpu.async_copy(
                table_ref.at[idx_v.at[j]],
                rows_v.at[pl.ds(j * IDX_CHUNK, IDX_CHUNK)],
                sem,
            )
            for j in range(K)
        ]
        for cp in copies:
            cp.wait()
        pltpu.sync_copy(rows_v, out_ref.at[wid, c])


@jax.jit
def kernel(x, table):
    xf = x.reshape(NW, NCHUNK, K, IDX_CHUNK).astype(jnp.int32)
    mesh = plsc.VectorSubcoreMesh(
        core_axis_name="c", subcore_axis_name="s", num_cores=NC, num_subcores=NS
    )
    out = pl.kernel(
        _body,
        out_type=jax.ShapeDtypeStruct((NW, NCHUNK, CHUNK, EMBED_DIM), jnp.float32),
        mesh=mesh,
        scratch_types=[
            pltpu.VMEM((K, IDX_CHUNK), jnp.int32),
            pltpu.VMEM((CHUNK, EMBED_DIM), jnp.float32),
            pltpu.SemaphoreType.DMA,
        ],
        compiler_params=pltpu.CompilerParams(use_tc_tiling_on_sc=False),
    )(xf, table)
    return out.reshape(x.shape[0], x.shape[1], EMBED_DIM)


# speedup vs baseline: 1.4955x; 1.0156x over previous
import jax
import jax.numpy as jnp
from jax import lax
from jax.experimental import pallas as pl
from jax.experimental.pallas import tpu as pltpu
from jax.experimental.pallas import tpu_sc as plsc

VOCAB = 1000000
EMBED_DIM = 32

NC = 2
NS = 16
NW = NC * NS

B_TOTAL = 4096 * 200
PER_W = B_TOTAL // NW
IDX_CHUNK = 128
K = 10
CHUNK = IDX_CHUNK * K
NCHUNK = PER_W // CHUNK
NBUF = 2


def _body(x_ref, table_ref, out_ref, idx0, idx1, rows0, rows1,
          isem0, isem1, gsem0, gsem1, osem0, osem1):
    wid = lax.axis_index("s") * NC + lax.axis_index("c")
    idx_v = (idx0, idx1)
    rows_v = (rows0, rows1)
    isem = (isem0, isem1)
    gsem = (gsem0, gsem1)
    osem = (osem0, osem1)

    for b in range(NBUF):
        pltpu.async_copy(x_ref.at[wid, b], idx_v[b], isem[b])

    @pl.loop(0, NCHUNK, step=NBUF)
    def _chunk(c0):
        for b in range(NBUF):
            c = c0 + b
            pltpu.make_async_copy(x_ref.at[wid, c], idx_v[b], isem[b]).wait()
            @pl.when(c >= NBUF)
            def _():
                pltpu.make_async_copy(
                    rows_v[b], out_ref.at[wid, c], osem[b]
                ).wait()

            copies = [
                pltpu.async_copy(
                    table_ref.at[idx_v[b].at[j]],
                    rows_v[b].at[pl.ds(j * IDX_CHUNK, IDX_CHUNK)],
                    gsem[b],
                )
                for j in range(K)
            ]

            for cp in copies:
                cp.wait()

            @pl.when(c + NBUF < NCHUNK)
            def _():
                pltpu.async_copy(x_ref.at[wid, c + NBUF], idx_v[b], isem[b])

            pltpu.async_copy(rows_v[b], out_ref.at[wid, c], osem[b])

    for b in range(NBUF):
        pltpu.make_async_copy(
            rows_v[b], out_ref.at[wid, NCHUNK - NBUF + b], osem[b]
        ).wait()


@jax.jit
def kernel(x, table):
    xf = x.reshape(NW, NCHUNK, K, IDX_CHUNK).astype(jnp.int32)
    mesh = plsc.VectorSubcoreMesh(
        core_axis_name="c", subcore_axis_name="s", num_cores=NC, num_subcores=NS
    )
    out = pl.kernel(
        _body,
        out_type=jax.ShapeDtypeStruct((NW, NCHUNK, CHUNK, EMBED_DIM), jnp.float32),
        mesh=mesh,
        scratch_types=[
            pltpu.VMEM((K, IDX_CHUNK), jnp.int32),
            pltpu.VMEM((K, IDX_CHUNK), jnp.int32),
            pltpu.VMEM((CHUNK, EMBED_DIM), jnp.float32),
            pltpu.VMEM((CHUNK, EMBED_DIM), jnp.float32),
            pltpu.SemaphoreType.DMA,
            pltpu.SemaphoreType.DMA,
            pltpu.SemaphoreType.DMA,
            pltpu.SemaphoreType.DMA,
            pltpu.SemaphoreType.DMA,
            pltpu.SemaphoreType.DMA,
        ],
        compiler_params=pltpu.CompilerParams(use_tc_tiling_on_sc=False),
    )(xf, table)
    return out.reshape(x.shape[0], x.shape[1], EMBED_DIM)
